# baseline (speedup 1.0000x reference)
import functools

import numpy as np
import jax
import jax.numpy as jnp
from jax.experimental import pallas as pl
from jax.experimental.pallas import tpu as pltpu

_B = 128
_N = 2
_S = 256
_D = 16
_DFF = 32
_NF = _S // 2 + 1
_BN = _B * _N
_PATCHES = ((2, 4, 8), (4, 8, 16), (2, 4, 8), (2, 4, 8))
_LAYERS = ("enc1", "enc2", "dec1", "dec2")
_HIGH = jax.lax.Precision.HIGHEST


@functools.lru_cache(maxsize=1)
def _const_mats():
    t = np.arange(_S, dtype=np.float64)
    k = np.arange(_NF, dtype=np.float64)
    ang = 2.0 * np.pi * np.outer(t, k) / _S
    cos_f = np.cos(ang)
    sin_f = -np.sin(ang)
    w = np.ones(_NF)
    w[1:_NF - 1] = 2.0
    angi = 2.0 * np.pi * np.outer(k, t) / _S
    inv_c = (w[:, None] * np.cos(angi)) / _S
    inv_s = -(w[:, None] * np.sin(angi)) / _S
    inv_s[0, :] = 0.0
    inv_s[_NF - 1, :] = 0.0
    ma = np.zeros((_S, _S))
    for s in range(_S):
        idx = np.clip(np.arange(s - 12, s + 13), 0, _S - 1)
        np.add.at(ma[s], idx, 1.0 / 25.0)
    f32 = lambda a: jnp.asarray(a, jnp.float32)
    return f32(ma.T), f32(cos_f), f32(sin_f), f32(inv_c), f32(inv_s)


@functools.lru_cache(maxsize=4)
def _masks_for(patches):
    s = np.arange(_S)
    out = []
    for p in patches:
        same = (s[:, None] // p) == (s[None, :] // p)
        out.append(np.where(same, 0.0, -1e9))
    return jnp.asarray(np.stack(out, 0), jnp.float32)


def _ln(x, g, b):
    m = jnp.mean(x, axis=-1, keepdims=True)
    xc = x - m
    v = jnp.mean(xc * xc, axis=-1, keepdims=True)
    return xc * jax.lax.rsqrt(v + 1e-5) * g + b


def _ln_mxu(x, g, b):
    ones16 = jnp.full((_D, _D), 1.0 / _D, jnp.float32)
    m = jnp.dot(x, ones16, preferred_element_type=jnp.float32)
    xc = x - m
    v = jnp.dot(xc * xc, ones16, preferred_element_type=jnp.float32)
    return xc * jax.lax.rsqrt(v + 1e-5) * g + b


def _gate_kernel(two_xg, xg_ref, xg2_ref, ab_ref, ma_ref, cf_ref, sf_ref,
                 ic_ref, is_ref, sw_ref, sb_ref, wg_ref,
                 eid_out, ord_out):
    xg = xg_ref[...].reshape(_BN, _S) * ab_ref[0, 0] + ab_ref[0, 1]
    if two_xg:
        xg = xg + xg2_ref[...].reshape(_BN, _S)
    trend = jnp.dot(xg, ma_ref[...], precision=_HIGH,
                    preferred_element_type=jnp.float32)
    fr = jnp.dot(xg, cf_ref[...], precision=_HIGH,
                 preferred_element_type=jnp.float32)
    fi = jnp.dot(xg, sf_ref[...], precision=_HIGH,
                 preferred_element_type=jnp.float32)
    amp = jnp.sqrt(fr * fr + fi * fi)
    kidx = jax.lax.broadcasted_iota(jnp.int32, (_BN, _NF), 1)
    amp = jnp.where(kidx == 0, 0.0, amp)
    work = amp
    cnt = jnp.zeros((_BN, 1), jnp.float32)
    thr = jnp.zeros((_BN, 1), jnp.float32)
    for _ in range(4):
        m = jnp.max(work, axis=1, keepdims=True)
        thr = jnp.where(cnt < 4.0, m, thr)
        cnt = cnt + jnp.sum((work == m).astype(jnp.float32), axis=1,
                            keepdims=True)
        work = jnp.where(work == m, -1.0, work)
    fmask = (amp >= thr).astype(jnp.float32)
    season = (jnp.dot(fr * fmask, ic_ref[...], precision=_HIGH,
                      preferred_element_type=jnp.float32)
              + jnp.dot(fi * fmask, is_ref[...], precision=_HIGH,
                        preferred_element_type=jnp.float32))
    nx = (xg + season + trend).reshape(_B, _N, _S)
    g = nx[:, 0, :] * sw_ref[0, 0] + nx[:, 1, :] * sw_ref[0, 1] + sb_ref[0, 0]
    logits = jnp.dot(g, wg_ref[...], precision=_HIGH,
                     preferred_element_type=jnp.float32)
    l0, l1_, l2_ = logits[:, 0:1], logits[:, 1:2], logits[:, 2:3]
    e0 = jnp.logical_and(l0 >= l1_, l0 >= l2_)
    e1 = jnp.logical_and(jnp.logical_not(e0), l1_ >= l2_)
    e0f = e0.astype(jnp.float32)
    e1f = e1.astype(jnp.float32)
    e2f = 1.0 - e0f - e1f
    eid = e1f + 2.0 * e2f
    bcolf = jax.lax.broadcasted_iota(jnp.int32, (_B, 1), 0).astype(jnp.float32)
    key = eid * float(_B) + bcolf
    key_row = jnp.transpose(key)
    less = (key_row < key).astype(jnp.float32)
    rank = jnp.sum(less, axis=1, keepdims=True)
    icol = jax.lax.broadcasted_iota(jnp.int32, (_B, _B), 1).astype(jnp.float32)
    bmat = jax.lax.broadcasted_iota(jnp.int32, (_B, _B), 0).astype(jnp.float32)
    onehot = (rank == icol).astype(jnp.float32)
    ordf = jnp.sum(onehot * bmat, axis=0, keepdims=True)
    c0 = jnp.sum(e0f, axis=0, keepdims=True)
    c01 = c0 + jnp.sum(e1f, axis=0, keepdims=True)
    irow = jax.lax.broadcasted_iota(jnp.int32, (1, _B), 1).astype(jnp.float32)
    esrt = (irow >= c0).astype(jnp.float32) + (irow >= c01).astype(jnp.float32)
    eid_out[...] = esrt.astype(jnp.int32)
    ord_out[...] = ordf.astype(jnp.int32)


def _gate_call(xg, xg2, ab, sw, sb, wg):
    ma_t, cos_f, sin_f, inv_c, inv_s = _const_mats()
    two = xg2 is not None
    ops = [xg] + ([xg2] if two else [xg]) + [ab, ma_t, cos_f, sin_f,
                                            inv_c, inv_s, sw, sb, wg]
    full = lambda a: pl.BlockSpec(a.shape, lambda i: (0,) * a.ndim)
    eid_s, order = pl.pallas_call(
        functools.partial(_gate_kernel, two),
        grid=(1,),
        in_specs=[full(a) for a in ops],
        out_specs=[pl.BlockSpec((1, _B), lambda i: (0, 0))] * 2,
        out_shape=[jax.ShapeDtypeStruct((1, _B), jnp.int32)] * 2,
    )(*ops)
    return eid_s.reshape(_B), order.reshape(_B)


def _expert_kernel(first, add_skip, final, ord_s, eid_s, h_ref, *rest):
    if add_skip:
        skip_ref, rest = rest[0], rest[1:]
    (mask_ref, wq_ref, wk_ref, wv_ref, wo_ref, w1_ref, b1_ref,
     w2_ref, b2_ref, l1g_ref, l1b_ref, l2g_ref, l2b_ref,
     w0_ref, b0_ref, wout_ref, bout_ref) = rest[:17]
    outs = rest[17:]
    if first:
        xb = h_ref[...].reshape(_N, _S)
        h2 = xb[:, :, None] * w0_ref[0][None, None, :] + b0_ref[0][None, None, :]
    else:
        h2 = h_ref[...]
    if add_skip:
        h2 = h2 + skip_ref[...]
    hf = h2.reshape(_N * _S, _D)
    q = jnp.dot(hf, wq_ref[0], preferred_element_type=jnp.float32)
    k = jnp.dot(hf, wk_ref[0], preferred_element_type=jnp.float32)
    v = jnp.dot(hf, wv_ref[0], preferred_element_type=jnp.float32)
    q = q.reshape(_N, _S, _D)
    k = k.reshape(_N, _S, _D)
    v = v.reshape(_N, _S, _D)
    att = jax.lax.dot_general(q, k, (((2,), (2,)), ((0,), (0,))),
                              preferred_element_type=jnp.float32)
    att = att * 0.25 + mask_ref[0][None, :, :]
    amax = jnp.max(att, axis=-1, keepdims=True)
    p = jnp.exp(att - amax)
    o = jax.lax.dot_general(p, v, (((2,), (1,)), ((0,), (0,))),
                            preferred_element_type=jnp.float32)
    den = jnp.dot(p.reshape(_N * _S, _S), jnp.ones((_S, _D), jnp.float32),
                  preferred_element_type=jnp.float32)
    o = o.reshape(_N * _S, _D) / den
    o = jnp.dot(o, wo_ref[0], preferred_element_type=jnp.float32)
    t = _ln_mxu(hf + o, l1g_ref[0], l1b_ref[0])
    f = jnp.dot(jax.nn.relu(jnp.dot(t, w1_ref[0],
                                    preferred_element_type=jnp.float32)
                            + b1_ref[0]),
                w2_ref[0], preferred_element_type=jnp.float32) + b2_ref[0]
    out = h2 + _ln_mxu(t + f, l2g_ref[0], l2b_ref[0]).reshape(_N, _S, _D)
    if final:
        y = jnp.sum(out * wout_ref[0][None, None, :], axis=-1) + bout_ref[0, 0]
        outs[0][...] = y.reshape(1, _N, _S)
    else:
        outs[0][...] = out
        outs[1][...] = out[:, :, 0].reshape(1, _N, _S)


def _expert_call(layer, first, add_skip, final, h, skip, eid_s, order,
                 masks3, ew, w0, b0, wout, bout):
    wq3, wk3, wv3, wo3, w13, b13, w23, b23, g13, bb13, g23, bb23 = ew

    bsel = lambda i, o, e: (o[i], 0, 0)
    esel = lambda i, o, e: (e[i], 0, 0)
    hsel = lambda i, o, e: (o[i], 0, 0)
    h_spec = (pl.BlockSpec((1, _N, _S), bsel) if first
              else pl.BlockSpec((_N, _S, _D), hsel))
    in_specs = [h_spec]
    operands = [h]
    if skip is not None:
        in_specs.append(pl.BlockSpec((_N, _S, _D), hsel))
        operands.append(skip)
    in_specs += [
        pl.BlockSpec((1, _S, _S), esel),
        pl.BlockSpec((1, _D, _D), esel), pl.BlockSpec((1, _D, _D), esel),
        pl.BlockSpec((1, _D, _D), esel), pl.BlockSpec((1, _D, _D), esel),
        pl.BlockSpec((1, _D, _DFF), esel), pl.BlockSpec((1, 1, _DFF), esel),
        pl.BlockSpec((1, _DFF, _D), esel), pl.BlockSpec((1, 1, _D), esel),
        pl.BlockSpec((1, 1, _D), esel), pl.BlockSpec((1, 1, _D), esel),
        pl.BlockSpec((1, 1, _D), esel), pl.BlockSpec((1, 1, _D), esel),
        pl.BlockSpec((1, _D), lambda i, o, e: (0, 0)),
        pl.BlockSpec((1, _D), lambda i, o, e: (0, 0)),
        pl.BlockSpec((1, _D), lambda i, o, e: (0, 0)),
        pl.BlockSpec((1, 1), lambda i, o, e: (0, 0)),
    ]
    operands += [masks3, wq3, wk3, wv3, wo3, w13, b13, w23, b23,
                 g13, bb13, g23, bb23, w0, b0, wout, bout]
    if final:
        out_specs = [pl.BlockSpec((1, _N, _S), bsel)]
        out_shape = [jax.ShapeDtypeStruct((_B, _N, _S), jnp.float32)]
    else:
        out_specs = [pl.BlockSpec((_N, _S, _D), hsel),
                     pl.BlockSpec((1, _N, _S), bsel)]
        out_shape = [jax.ShapeDtypeStruct((_BN, _S, _D), jnp.float32),
                     jax.ShapeDtypeStruct((_B, _N, _S), jnp.float32)]
    grid_spec = pltpu.PrefetchScalarGridSpec(
        num_scalar_prefetch=2,
        grid=(_B,),
        in_specs=in_specs,
        out_specs=out_specs,
    )
    return pl.pallas_call(
        functools.partial(_expert_kernel, first, add_skip, final),
        grid_spec=grid_spec,
        out_shape=out_shape,
        compiler_params=pltpu.CompilerParams(
            dimension_semantics=("parallel",)),
    )(order, eid_s, *operands)


def kernel(x, params):
    w0 = params["start_fc_w"].reshape(1, _D)
    b0 = params["start_fc_b"].reshape(1, _D)
    wout = params["out_fc_w"].reshape(1, _D)
    bout = params["out_fc_b"].reshape(1, 1)

    def layer_weights(name):
        p = params[name]
        ew = []
        for key, shp in (("Wq", None), ("Wk", None), ("Wv", None), ("Wo", None),
                         ("W1", None), ("b1", (1, _DFF)), ("W2", None),
                         ("b2", (1, _D)), ("ln1_g", (1, _D)), ("ln1_b", (1, _D)),
                         ("ln2_g", (1, _D)), ("ln2_b", (1, _D))):
            arrs = [p["experts"][e][key] for e in range(3)]
            if shp is not None:
                arrs = [a.reshape(shp) for a in arrs]
            ew.append(jnp.stack(arrs, axis=0))
        gw = (jnp.asarray(1.0, jnp.float32),
              p["start_w"].reshape(1, _N), p["start_b"].reshape(1, 1),
              p["w_gate"])
        return ew, gw

    x_rows = x.reshape(_B, _N, _S)
    one = jnp.ones((1, 1), jnp.float32)
    ab_first = jnp.concatenate(
        [w0[:, 0:1], b0[:, 0:1]], axis=1)
    ab_id = jnp.concatenate([one, 0.0 * one], axis=1)

    h = None
    xg = x_rows
    x1 = None
    xg1 = None
    for li, name in enumerate(_LAYERS):
        ew, (_, sw, sb, wg) = layer_weights(name)
        masks3 = _masks_for(_PATCHES[li])
        first = li == 0
        final = li == 3
        ab = ab_first if first else ab_id
        xg2 = xg1 if li == 3 else None
        eid_s, order = _gate_call(xg, xg2, ab, sw, sb, wg)
        src = x_rows if first else h
        skip = x1 if li == 3 else None
        res = _expert_call(li, first, skip is not None, final, src, skip,
                           eid_s, order, masks3, ew, w0, b0, wout, bout)
        if final:
            y = res[0]
        else:
            h, xg = res
            if li == 0:
                x1, xg1 = h, xg
    return y, jnp.asarray(0.0, jnp.float32)


# speedup vs baseline: 1.0599x; 1.0599x over previous
import functools

import numpy as np
import jax
import jax.numpy as jnp
from jax.experimental import pallas as pl
from jax.experimental.pallas import tpu as pltpu

_B = 128
_N = 2
_S = 256
_D = 16
_DFF = 32
_NF = _S // 2 + 1
_BN = _B * _N
_PATCHES = ((2, 4, 8), (4, 8, 16), (2, 4, 8), (2, 4, 8))
_LAYERS = ("enc1", "enc2", "dec1", "dec2")
_HIGH = jax.lax.Precision.HIGHEST


@functools.lru_cache(maxsize=1)
def _const_mats():
    t = np.arange(_S, dtype=np.float64)
    k = np.arange(_NF, dtype=np.float64)
    ang = 2.0 * np.pi * np.outer(t, k) / _S
    cos_f = np.cos(ang)
    sin_f = -np.sin(ang)
    w = np.ones(_NF)
    w[1:_NF - 1] = 2.0
    angi = 2.0 * np.pi * np.outer(k, t) / _S
    inv_c = (w[:, None] * np.cos(angi)) / _S
    inv_s = -(w[:, None] * np.sin(angi)) / _S
    inv_s[0, :] = 0.0
    inv_s[_NF - 1, :] = 0.0
    ma = np.zeros((_S, _S))
    for s in range(_S):
        idx = np.clip(np.arange(s - 12, s + 13), 0, _S - 1)
        np.add.at(ma[s], idx, 1.0 / 25.0)
    f32 = lambda a: jnp.asarray(a, jnp.float32)
    return f32(ma.T), f32(cos_f), f32(sin_f), f32(inv_c), f32(inv_s)


@functools.lru_cache(maxsize=4)
def _masks_for(patches):
    s = np.arange(_S)
    out = []
    for p in patches:
        same = (s[:, None] // p) == (s[None, :] // p)
        out.append(np.where(same, 0.0, -1e9))
    return jnp.asarray(np.stack(out, 0), jnp.float32)


def _ln(x, g, b):
    m = jnp.mean(x, axis=-1, keepdims=True)
    xc = x - m
    v = jnp.mean(xc * xc, axis=-1, keepdims=True)
    return xc * jax.lax.rsqrt(v + 1e-5) * g + b


def _gate_kernel(two_xg, xg_ref, xg2_ref, ab_ref, ma_ref, cf_ref, sf_ref,
                 ic_ref, is_ref, sw_ref, sb_ref, wg_ref,
                 eid_out, ord_out):
    xg = xg_ref[...].reshape(_BN, _S) * ab_ref[0, 0] + ab_ref[0, 1]
    if two_xg:
        xg = xg + xg2_ref[...].reshape(_BN, _S)
    trend = jnp.dot(xg, ma_ref[...], precision=_HIGH,
                    preferred_element_type=jnp.float32)
    fr = jnp.dot(xg, cf_ref[...], precision=_HIGH,
                 preferred_element_type=jnp.float32)
    fi = jnp.dot(xg, sf_ref[...], precision=_HIGH,
                 preferred_element_type=jnp.float32)
    amp = jnp.sqrt(fr * fr + fi * fi)
    kidx = jax.lax.broadcasted_iota(jnp.int32, (_BN, _NF), 1)
    amp = jnp.where(kidx == 0, 0.0, amp)
    work = amp
    cnt = jnp.zeros((_BN, 1), jnp.float32)
    thr = jnp.zeros((_BN, 1), jnp.float32)
    for _ in range(4):
        m = jnp.max(work, axis=1, keepdims=True)
        thr = jnp.where(cnt < 4.0, m, thr)
        cnt = cnt + jnp.sum((work == m).astype(jnp.float32), axis=1,
                            keepdims=True)
        work = jnp.where(work == m, -1.0, work)
    fmask = (amp >= thr).astype(jnp.float32)
    season = (jnp.dot(fr * fmask, ic_ref[...], precision=_HIGH,
                      preferred_element_type=jnp.float32)
              + jnp.dot(fi * fmask, is_ref[...], precision=_HIGH,
                        preferred_element_type=jnp.float32))
    nx = (xg + season + trend).reshape(_B, _N, _S)
    g = nx[:, 0, :] * sw_ref[0, 0] + nx[:, 1, :] * sw_ref[0, 1] + sb_ref[0, 0]
    logits = jnp.dot(g, wg_ref[...], precision=_HIGH,
                     preferred_element_type=jnp.float32)
    l0, l1_, l2_ = logits[:, 0:1], logits[:, 1:2], logits[:, 2:3]
    e0 = jnp.logical_and(l0 >= l1_, l0 >= l2_)
    e1 = jnp.logical_and(jnp.logical_not(e0), l1_ >= l2_)
    e0f = e0.astype(jnp.float32)
    e1f = e1.astype(jnp.float32)
    e2f = 1.0 - e0f - e1f
    eid = e1f + 2.0 * e2f
    bcolf = jax.lax.broadcasted_iota(jnp.int32, (_B, 1), 0).astype(jnp.float32)
    key = eid * float(_B) + bcolf
    key_row = jnp.transpose(key)
    less = (key_row < key).astype(jnp.float32)
    rank = jnp.sum(less, axis=1, keepdims=True)
    icol = jax.lax.broadcasted_iota(jnp.int32, (_B, _B), 1).astype(jnp.float32)
    bmat = jax.lax.broadcasted_iota(jnp.int32, (_B, _B), 0).astype(jnp.float32)
    onehot = (rank == icol).astype(jnp.float32)
    ordf = jnp.sum(onehot * bmat, axis=0, keepdims=True)
    c0 = jnp.sum(e0f, axis=0, keepdims=True)
    c01 = c0 + jnp.sum(e1f, axis=0, keepdims=True)
    irow = jax.lax.broadcasted_iota(jnp.int32, (1, _B), 1).astype(jnp.float32)
    esrt = (irow >= c0).astype(jnp.float32) + (irow >= c01).astype(jnp.float32)
    eid_out[...] = esrt.astype(jnp.int32)
    ord_out[...] = ordf.astype(jnp.int32)


def _gate_call(xg, xg2, ab, sw, sb, wg):
    ma_t, cos_f, sin_f, inv_c, inv_s = _const_mats()
    two = xg2 is not None
    ops = [xg] + ([xg2] if two else [xg]) + [ab, ma_t, cos_f, sin_f,
                                            inv_c, inv_s, sw, sb, wg]
    full = lambda a: pl.BlockSpec(a.shape, lambda i: (0,) * a.ndim)
    eid_s, order = pl.pallas_call(
        functools.partial(_gate_kernel, two),
        grid=(1,),
        in_specs=[full(a) for a in ops],
        out_specs=[pl.BlockSpec((1, _B), lambda i: (0, 0))] * 2,
        out_shape=[jax.ShapeDtypeStruct((1, _B), jnp.int32)] * 2,
    )(*ops)
    return eid_s.reshape(_B), order.reshape(_B)


def _expert_kernel(first, add_skip, final, ord_s, eid_s, h_ref, *rest):
    if add_skip:
        skip_ref, rest = rest[0], rest[1:]
    (mask_ref, wq_ref, wk_ref, wv_ref, wo_ref, w1_ref, b1_ref,
     w2_ref, b2_ref, l1g_ref, l1b_ref, l2g_ref, l2b_ref,
     w0_ref, b0_ref, wout_ref, bout_ref) = rest[:17]
    outs = rest[17:]
    if first:
        xb = h_ref[...].reshape(_N, _S)
        h2 = xb[:, :, None] * w0_ref[0][None, None, :] + b0_ref[0][None, None, :]
    else:
        h2 = h_ref[...]
    if add_skip:
        h2 = h2 + skip_ref[...]
    hf = h2.reshape(_N * _S, _D)
    q = jnp.dot(hf, wq_ref[0] * 0.25, preferred_element_type=jnp.float32)
    k = jnp.dot(hf, wk_ref[0], preferred_element_type=jnp.float32)
    v = jnp.dot(hf, wv_ref[0], preferred_element_type=jnp.float32)
    q = q.reshape(_N, _S, _D)
    k = k.reshape(_N, _S, _D)
    v = v.reshape(_N, _S, _D)
    att = jax.lax.dot_general(q, k, (((2,), (2,)), ((0,), (0,))),
                              preferred_element_type=jnp.float32)
    att = att + mask_ref[0][None, :, :]
    att = jax.nn.softmax(att, axis=-1)
    o = jax.lax.dot_general(att, v, (((2,), (1,)), ((0,), (0,))),
                            preferred_element_type=jnp.float32)
    o = jnp.dot(o.reshape(_N * _S, _D), wo_ref[0],
                preferred_element_type=jnp.float32)
    t = _ln(hf + o, l1g_ref[0], l1b_ref[0])
    f = jnp.dot(jax.nn.relu(jnp.dot(t, w1_ref[0],
                                    preferred_element_type=jnp.float32)
                            + b1_ref[0]),
                w2_ref[0], preferred_element_type=jnp.float32) + b2_ref[0]
    out = h2 + _ln(t + f, l2g_ref[0], l2b_ref[0]).reshape(_N, _S, _D)
    if final:
        y = jnp.sum(out * wout_ref[0][None, None, :], axis=-1) + bout_ref[0, 0]
        outs[0][...] = y.reshape(1, _N, _S)
    else:
        outs[0][...] = out
        outs[1][...] = out[:, :, 0].reshape(1, _N, _S)


def _expert_call(layer, first, add_skip, final, h, skip, eid_s, order,
                 masks3, ew, w0, b0, wout, bout):
    wq3, wk3, wv3, wo3, w13, b13, w23, b23, g13, bb13, g23, bb23 = ew

    bsel = lambda i, o, e: (o[i], 0, 0)
    esel = lambda i, o, e: (e[i], 0, 0)
    hsel = lambda i, o, e: (o[i], 0, 0)
    h_spec = (pl.BlockSpec((1, _N, _S), bsel) if first
              else pl.BlockSpec((_N, _S, _D), hsel))
    in_specs = [h_spec]
    operands = [h]
    if skip is not None:
        in_specs.append(pl.BlockSpec((_N, _S, _D), hsel))
        operands.append(skip)
    in_specs += [
        pl.BlockSpec((1, _S, _S), esel),
        pl.BlockSpec((1, _D, _D), esel), pl.BlockSpec((1, _D, _D), esel),
        pl.BlockSpec((1, _D, _D), esel), pl.BlockSpec((1, _D, _D), esel),
        pl.BlockSpec((1, _D, _DFF), esel), pl.BlockSpec((1, 1, _DFF), esel),
        pl.BlockSpec((1, _DFF, _D), esel), pl.BlockSpec((1, 1, _D), esel),
        pl.BlockSpec((1, 1, _D), esel), pl.BlockSpec((1, 1, _D), esel),
        pl.BlockSpec((1, 1, _D), esel), pl.BlockSpec((1, 1, _D), esel),
        pl.BlockSpec((1, _D), lambda i, o, e: (0, 0)),
        pl.BlockSpec((1, _D), lambda i, o, e: (0, 0)),
        pl.BlockSpec((1, _D), lambda i, o, e: (0, 0)),
        pl.BlockSpec((1, 1), lambda i, o, e: (0, 0)),
    ]
    operands += [masks3, wq3, wk3, wv3, wo3, w13, b13, w23, b23,
                 g13, bb13, g23, bb23, w0, b0, wout, bout]
    if final:
        out_specs = [pl.BlockSpec((1, _N, _S), bsel)]
        out_shape = [jax.ShapeDtypeStruct((_B, _N, _S), jnp.float32)]
    else:
        out_specs = [pl.BlockSpec((_N, _S, _D), hsel),
                     pl.BlockSpec((1, _N, _S), bsel)]
        out_shape = [jax.ShapeDtypeStruct((_BN, _S, _D), jnp.float32),
                     jax.ShapeDtypeStruct((_B, _N, _S), jnp.float32)]
    grid_spec = pltpu.PrefetchScalarGridSpec(
        num_scalar_prefetch=2,
        grid=(_B,),
        in_specs=in_specs,
        out_specs=out_specs,
    )
    return pl.pallas_call(
        functools.partial(_expert_kernel, first, add_skip, final),
        grid_spec=grid_spec,
        out_shape=out_shape,
        compiler_params=pltpu.CompilerParams(
            dimension_semantics=("parallel",)),
    )(order, eid_s, *operands)


def kernel(x, params):
    w0 = params["start_fc_w"].reshape(1, _D)
    b0 = params["start_fc_b"].reshape(1, _D)
    wout = params["out_fc_w"].reshape(1, _D)
    bout = params["out_fc_b"].reshape(1, 1)

    def layer_weights(name):
        p = params[name]
        ew = []
        for key, shp in (("Wq", None), ("Wk", None), ("Wv", None), ("Wo", None),
                         ("W1", None), ("b1", (1, _DFF)), ("W2", None),
                         ("b2", (1, _D)), ("ln1_g", (1, _D)), ("ln1_b", (1, _D)),
                         ("ln2_g", (1, _D)), ("ln2_b", (1, _D))):
            arrs = [p["experts"][e][key] for e in range(3)]
            if shp is not None:
                arrs = [a.reshape(shp) for a in arrs]
            ew.append(jnp.stack(arrs, axis=0))
        gw = (jnp.asarray(1.0, jnp.float32),
              p["start_w"].reshape(1, _N), p["start_b"].reshape(1, 1),
              p["w_gate"])
        return ew, gw

    x_rows = x.reshape(_B, _N, _S)
    one = jnp.ones((1, 1), jnp.float32)
    ab_first = jnp.concatenate(
        [w0[:, 0:1], b0[:, 0:1]], axis=1)
    ab_id = jnp.concatenate([one, 0.0 * one], axis=1)

    h = None
    xg = x_rows
    x1 = None
    xg1 = None
    for li, name in enumerate(_LAYERS):
        ew, (_, sw, sb, wg) = layer_weights(name)
        masks3 = _masks_for(_PATCHES[li])
        first = li == 0
        final = li == 3
        ab = ab_first if first else ab_id
        xg2 = xg1 if li == 3 else None
        eid_s, order = _gate_call(xg, xg2, ab, sw, sb, wg)
        src = x_rows if first else h
        skip = x1 if li == 3 else None
        res = _expert_call(li, first, skip is not None, final, src, skip,
                           eid_s, order, masks3, ew, w0, b0, wout, bout)
        if final:
            y = res[0]
        else:
            h, xg = res
            if li == 0:
                x1, xg1 = h, xg
    return y, jnp.asarray(0.0, jnp.float32)


# speedup vs baseline: 1.0860x; 1.0247x over previous
import functools

import numpy as np
import jax
import jax.numpy as jnp
from jax.experimental import pallas as pl
from jax.experimental.pallas import tpu as pltpu

_B = 128
_N = 2
_S = 256
_D = 16
_DFF = 32
_NF = _S // 2 + 1
_BN = _B * _N
_PATCHES = ((2, 4, 8), (4, 8, 16), (2, 4, 8), (2, 4, 8))
_LAYERS = ("enc1", "enc2", "dec1", "dec2")
_HIGH = jax.lax.Precision.HIGHEST


@functools.lru_cache(maxsize=1)
def _const_mats():
    t = np.arange(_S, dtype=np.float64)
    k = np.arange(_NF, dtype=np.float64)
    ang = 2.0 * np.pi * np.outer(t, k) / _S
    cos_f = np.cos(ang)
    sin_f = -np.sin(ang)
    w = np.ones(_NF)
    w[1:_NF - 1] = 2.0
    angi = 2.0 * np.pi * np.outer(k, t) / _S
    inv_c = (w[:, None] * np.cos(angi)) / _S
    inv_s = -(w[:, None] * np.sin(angi)) / _S
    inv_s[0, :] = 0.0
    inv_s[_NF - 1, :] = 0.0
    ma = np.zeros((_S, _S))
    for s in range(_S):
        idx = np.clip(np.arange(s - 12, s + 13), 0, _S - 1)
        np.add.at(ma[s], idx, 1.0 / 25.0)
    f32 = lambda a: jnp.asarray(a, jnp.float32)
    return f32(ma.T), f32(cos_f), f32(sin_f), f32(inv_c), f32(inv_s)


@functools.lru_cache(maxsize=4)
def _masks_for(patches):
    s = np.arange(_S)
    out = []
    for p in patches:
        same = (s[:, None] // p) == (s[None, :] // p)
        out.append(np.where(same, 0.0, -1e9))
    return jnp.asarray(np.stack(out, 0), jnp.float32)


def _ln(x, g, b):
    m = jnp.mean(x, axis=-1, keepdims=True)
    xc = x - m
    v = jnp.mean(xc * xc, axis=-1, keepdims=True)
    return xc * jax.lax.rsqrt(v + 1e-5) * g + b


def _gate_kernel(two_xg, xg_ref, xg2_ref, ab_ref, ma_ref, cf_ref, sf_ref,
                 ic_ref, is_ref, sw_ref, sb_ref, wg_ref,
                 eid_out, ord_out, par_out):
    xg = xg_ref[...].reshape(_BN, _S) * ab_ref[0, 0] + ab_ref[0, 1]
    if two_xg:
        xg = xg + xg2_ref[...].reshape(_BN, _S)
    trend = jnp.dot(xg, ma_ref[...], precision=_HIGH,
                    preferred_element_type=jnp.float32)
    fr = jnp.dot(xg, cf_ref[...], precision=_HIGH,
                 preferred_element_type=jnp.float32)
    fi = jnp.dot(xg, sf_ref[...], precision=_HIGH,
                 preferred_element_type=jnp.float32)
    amp = jnp.sqrt(fr * fr + fi * fi)
    kidx = jax.lax.broadcasted_iota(jnp.int32, (_BN, _NF), 1)
    amp = jnp.where(kidx == 0, 0.0, amp)
    work = amp
    cnt = jnp.zeros((_BN, 1), jnp.float32)
    thr = jnp.zeros((_BN, 1), jnp.float32)
    for _ in range(4):
        m = jnp.max(work, axis=1, keepdims=True)
        thr = jnp.where(cnt < 4.0, m, thr)
        cnt = cnt + jnp.sum((work == m).astype(jnp.float32), axis=1,
                            keepdims=True)
        work = jnp.where(work == m, -1.0, work)
    fmask = (amp >= thr).astype(jnp.float32)
    season = (jnp.dot(fr * fmask, ic_ref[...], precision=_HIGH,
                      preferred_element_type=jnp.float32)
              + jnp.dot(fi * fmask, is_ref[...], precision=_HIGH,
                        preferred_element_type=jnp.float32))
    nx = (xg + season + trend).reshape(_B, _N, _S)
    g = nx[:, 0, :] * sw_ref[0, 0] + nx[:, 1, :] * sw_ref[0, 1] + sb_ref[0, 0]
    logits = jnp.dot(g, wg_ref[...], precision=_HIGH,
                     preferred_element_type=jnp.float32)
    l0, l1_, l2_ = logits[:, 0:1], logits[:, 1:2], logits[:, 2:3]
    e0 = jnp.logical_and(l0 >= l1_, l0 >= l2_)
    e1 = jnp.logical_and(jnp.logical_not(e0), l1_ >= l2_)
    e0f = e0.astype(jnp.float32)
    e1f = e1.astype(jnp.float32)
    e2f = 1.0 - e0f - e1f
    eid = e1f + 2.0 * e2f
    bcolf = jax.lax.broadcasted_iota(jnp.int32, (_B, 1), 0).astype(jnp.float32)
    key = eid * float(_B) + bcolf
    key_row = jnp.transpose(key)
    less = (key_row < key).astype(jnp.float32)
    rank = jnp.sum(less, axis=1, keepdims=True)
    par = rank - 2.0 * jnp.floor(rank * 0.5)
    icol = jax.lax.broadcasted_iota(jnp.int32, (_B, _B), 1).astype(jnp.float32)
    bmat = jax.lax.broadcasted_iota(jnp.int32, (_B, _B), 0).astype(jnp.float32)
    onehot = (rank == icol).astype(jnp.float32)
    ordf = jnp.sum(onehot * bmat, axis=0, keepdims=True)
    c0 = jnp.sum(e0f, axis=0, keepdims=True)
    c01 = c0 + jnp.sum(e1f, axis=0, keepdims=True)
    irow = jax.lax.broadcasted_iota(jnp.int32, (1, _B), 1).astype(jnp.float32)
    esrt = (irow >= c0).astype(jnp.float32) + (irow >= c01).astype(jnp.float32)
    eid_out[...] = esrt.astype(jnp.int32)
    ord_out[...] = ordf.astype(jnp.int32)
    par_out[...] = jnp.transpose(par).astype(jnp.int32)


def _gate_call(xg, xg2, ab, sw, sb, wg):
    ma_t, cos_f, sin_f, inv_c, inv_s = _const_mats()
    two = xg2 is not None
    ops = [xg] + ([xg2] if two else [xg]) + [ab, ma_t, cos_f, sin_f,
                                            inv_c, inv_s, sw, sb, wg]
    full = lambda a: pl.BlockSpec(a.shape, lambda i: (0,) * a.ndim)
    eid_s, order, par = pl.pallas_call(
        functools.partial(_gate_kernel, two),
        grid=(1,),
        in_specs=[full(a) for a in ops],
        out_specs=[pl.BlockSpec((1, _B), lambda i: (0, 0))] * 3,
        out_shape=[jax.ShapeDtypeStruct((1, _B), jnp.int32)] * 3,
    )(*ops)
    return eid_s.reshape(_B), order.reshape(_B), par.reshape(_B)


def _expert_half(h2, maskr, wq, wk, wv, wo, w1, b1, w2, b2, g1, bb1, g2, bb2):
    hf = h2.reshape(_N * _S, _D)
    q = jnp.dot(hf, wq * 0.25, preferred_element_type=jnp.float32)
    k = jnp.dot(hf, wk, preferred_element_type=jnp.float32)
    v = jnp.dot(hf, wv, preferred_element_type=jnp.float32)
    q = q.reshape(_N, _S, _D)
    k = k.reshape(_N, _S, _D)
    v = v.reshape(_N, _S, _D)
    att = jax.lax.dot_general(q, k, (((2,), (2,)), ((0,), (0,))),
                              preferred_element_type=jnp.float32)
    att = att + maskr[None, :, :]
    att = jax.nn.softmax(att, axis=-1)
    o = jax.lax.dot_general(att, v, (((2,), (1,)), ((0,), (0,))),
                            preferred_element_type=jnp.float32)
    o = jnp.dot(o.reshape(_N * _S, _D), wo, preferred_element_type=jnp.float32)
    t = _ln(hf + o, g1, bb1)
    f = jnp.dot(jax.nn.relu(jnp.dot(t, w1, preferred_element_type=jnp.float32)
                            + b1),
                w2, preferred_element_type=jnp.float32) + b2
    return h2 + _ln(t + f, g2, bb2).reshape(_N, _S, _D)


def _expert_kernel(first, add_skip, final, ord_s, eid_s, ha_ref, hb_ref,
                   *rest):
    if add_skip:
        (skipa_ref, skipb_ref), rest = rest[:2], rest[2:]
    wsa = rest[:13]
    wsb = rest[13:26]
    w0_ref, b0_ref, wout_ref, bout_ref = rest[26:30]
    outs = rest[30:]

    halves = []
    for h_ref, ws in ((ha_ref, wsa), (hb_ref, wsb)):
        if first:
            xb = h_ref[...].reshape(_N, _S)
            h2 = (xb[:, :, None] * w0_ref[0][None, None, :]
                  + b0_ref[0][None, None, :])
        else:
            h2 = h_ref[...]
        halves.append(h2)
    if add_skip:
        halves[0] = halves[0] + skipa_ref[...]
        halves[1] = halves[1] + skipb_ref[...]

    for idx, ws in enumerate((wsa, wsb)):
        (mask_ref, wq_ref, wk_ref, wv_ref, wo_ref, w1_ref, b1_ref,
         w2_ref, b2_ref, l1g_ref, l1b_ref, l2g_ref, l2b_ref) = ws
        out = _expert_half(halves[idx], mask_ref[0],
                           wq_ref[0], wk_ref[0], wv_ref[0], wo_ref[0],
                           w1_ref[0], b1_ref[0], w2_ref[0], b2_ref[0],
                           l1g_ref[0], l1b_ref[0], l2g_ref[0], l2b_ref[0])
        if final:
            y = (jnp.sum(out * wout_ref[0][None, None, :], axis=-1)
                 + bout_ref[0, 0])
            outs[idx][...] = y.reshape(1, _N, _S)
        else:
            outs[2 * idx][...] = out
            outs[2 * idx + 1][...] = out[:, :, 0].reshape(1, _N, _S)


def _expert_call(first, add_skip, final, h, skip, eid_s, order,
                 masks3, ew, w0, b0, wout, bout):
    wq3, wk3, wv3, wo3, w13, b13, w23, b23, g13, bb13, g23, bb23 = ew

    def bsel(j):
        return lambda i, o, e: (o[2 * i + j], 0, 0)

    def esel(j):
        return lambda i, o, e: (e[2 * i + j], 0, 0)

    h_spec = lambda j: (pl.BlockSpec((1, _N, _S), bsel(j)) if first
                        else pl.BlockSpec((_N, _S, _D), bsel(j)))
    in_specs = [h_spec(0), h_spec(1)]
    operands = [h, h]
    if skip is not None:
        in_specs += [pl.BlockSpec((_N, _S, _D), bsel(0)),
                     pl.BlockSpec((_N, _S, _D), bsel(1))]
        operands += [skip, skip]
    wshapes = [(1, _S, _S), (1, _D, _D), (1, _D, _D), (1, _D, _D),
               (1, _D, _D), (1, _D, _DFF), (1, 1, _DFF), (1, _DFF, _D),
               (1, 1, _D), (1, 1, _D), (1, 1, _D), (1, 1, _D), (1, 1, _D)]
    warrs = [masks3, wq3, wk3, wv3, wo3, w13, b13, w23, b23,
             g13, bb13, g23, bb23]
    for j in (0, 1):
        in_specs += [pl.BlockSpec(s, esel(j)) for s in wshapes]
        operands += warrs
    in_specs += [pl.BlockSpec((1, _D), lambda i, o, e: (0, 0)),
                 pl.BlockSpec((1, _D), lambda i, o, e: (0, 0)),
                 pl.BlockSpec((1, _D), lambda i, o, e: (0, 0)),
                 pl.BlockSpec((1, 1), lambda i, o, e: (0, 0))]
    operands += [w0, b0, wout, bout]
    if final:
        out_specs = [pl.BlockSpec((1, _N, _S), bsel(0)),
                     pl.BlockSpec((1, _N, _S), bsel(1))]
        out_shape = [jax.ShapeDtypeStruct((_B, _N, _S), jnp.float32)] * 2
    else:
        out_specs = [pl.BlockSpec((_N, _S, _D), bsel(0)),
                     pl.BlockSpec((1, _N, _S), bsel(0)),
                     pl.BlockSpec((_N, _S, _D), bsel(1)),
                     pl.BlockSpec((1, _N, _S), bsel(1))]
        out_shape = [jax.ShapeDtypeStruct((_BN, _S, _D), jnp.float32),
                     jax.ShapeDtypeStruct((_B, _N, _S), jnp.float32)] * 2
    grid_spec = pltpu.PrefetchScalarGridSpec(
        num_scalar_prefetch=2,
        grid=(_B // 2,),
        in_specs=in_specs,
        out_specs=out_specs,
    )
    return pl.pallas_call(
        functools.partial(_expert_kernel, first, add_skip, final),
        grid_spec=grid_spec,
        out_shape=out_shape,
        compiler_params=pltpu.CompilerParams(
            dimension_semantics=("parallel",)),
    )(order, eid_s, *operands)


def kernel(x, params):
    w0 = params["start_fc_w"].reshape(1, _D)
    b0 = params["start_fc_b"].reshape(1, _D)
    wout = params["out_fc_w"].reshape(1, _D)
    bout = params["out_fc_b"].reshape(1, 1)

    def layer_weights(name):
        p = params[name]
        ew = []
        for key, shp in (("Wq", None), ("Wk", None), ("Wv", None), ("Wo", None),
                         ("W1", None), ("b1", (1, _DFF)), ("W2", None),
                         ("b2", (1, _D)), ("ln1_g", (1, _D)), ("ln1_b", (1, _D)),
                         ("ln2_g", (1, _D)), ("ln2_b", (1, _D))):
            arrs = [p["experts"][e][key] for e in range(3)]
            if shp is not None:
                arrs = [a.reshape(shp) for a in arrs]
            ew.append(jnp.stack(arrs, axis=0))
        gw = (p["start_w"].reshape(1, _N), p["start_b"].reshape(1, 1),
              p["w_gate"])
        return ew, gw

    x_rows = x.reshape(_B, _N, _S)
    one = jnp.ones((1, 1), jnp.float32)
    ab_first = jnp.concatenate([w0[:, 0:1], b0[:, 0:1]], axis=1)
    ab_id = jnp.concatenate([one, 0.0 * one], axis=1)

    h = None
    xg = x_rows
    x1 = None
    xg1 = None
    for li, name in enumerate(_LAYERS):
        ew, (sw, sb, wg) = layer_weights(name)
        masks3 = _masks_for(_PATCHES[li])
        first = li == 0
        final = li == 3
        ab = ab_first if first else ab_id
        xg2 = xg1 if li == 3 else None
        eid_s, order, par = _gate_call(xg, xg2, ab, sw, sb, wg)
        src = x_rows if first else h
        skip = x1 if li == 3 else None
        res = _expert_call(first, skip is not None, final, src, skip,
                           eid_s, order, masks3, ew, w0, b0, wout, bout)
        if final:
            ya, yb = res
            y = jnp.where((par == 1)[:, None, None], yb, ya)
        else:
            ha, xga, hb, xgb = res
            par_bn = jnp.repeat(par, _N)
            h = jnp.where((par_bn == 1)[:, None, None], hb, ha)
            xg = jnp.where((par == 1)[:, None, None], xgb, xga)
            if li == 0:
                x1, xg1 = h, xg
    return y, jnp.asarray(0.0, jnp.float32)
